# Initial kernel scaffold; baseline (speedup 1.0000x reference)
import jax
import jax.numpy as jnp
from jax.experimental import pallas as pl


def kernel(sphere_nodes, edge_features, senders, receivers, params):
    raise NotImplementedError("write your pallas kernel here")


# speedup vs baseline: 1.9783x; 1.9783x over previous
import functools

import jax
import jax.numpy as jnp
from jax import lax
from jax.experimental import pallas as pl
from jax.experimental.pallas import tpu as pltpu
from jax.experimental.pallas import tpu_sc as plsc

N_SPATIAL = 10000
N_SPHERE = 2883
D = 256
H = 128
E = 160000
NF = 78

NC, NS = 2, 16
NW = NC * NS

G_EPW = E // NW
G_C = 40
G_NCH = G_EPW // G_C

S_EPT = E // NS
S_C = 80
S_NCH = S_EPT // S_C
S_RPT = 624
S_TAIL = N_SPATIAL - NS * S_RPT

RE = 640
RN = 2000

_F32 = jnp.float32


def _ln(h, g, b):
    mu = jnp.mean(h, axis=1, keepdims=True)
    var = jnp.mean((h - mu) ** 2, axis=1, keepdims=True)
    return (h - mu) * lax.rsqrt(var + 1e-5) * g + b


def _full(shape):
    return pl.BlockSpec(shape, lambda i: (0,) * len(shape))


def _prep_body(sn, wp, ws, sp_out, as_out):
    sp = jnp.dot(sn[...], wp[...], preferred_element_type=_F32)
    sp_out[...] = sp
    as_out[...] = jnp.dot(sp, ws[...], preferred_element_type=_F32)


def _prep(sphere_nodes, wp, ws):
    return pl.pallas_call(
        _prep_body,
        out_shape=(jax.ShapeDtypeStruct((N_SPHERE, D), _F32),
                   jax.ShapeDtypeStruct((N_SPHERE, D), _F32)),
    )(sphere_nodes, wp, ws)


def _edge0_body(ef, g, w1, b1, gl, bl, w2, b2, o0, o1):
    x = jnp.dot(ef[...], w1[...], preferred_element_type=_F32) + g[...] + b1[...]
    h = _ln(jnp.maximum(x, 0.0), gl[...], bl[...])
    y = jnp.dot(h, w2[...], preferred_element_type=_F32) + b2[...]
    o0[...] = y[:, :H]
    o1[...] = y[:, H:]


def _edge0(ef, g, w1, b1, gl, bl, w2, b2):
    grid = (E // RE,)
    return pl.pallas_call(
        _edge0_body,
        grid=grid,
        in_specs=[
            pl.BlockSpec((RE, 3), lambda i: (i, 0)),
            pl.BlockSpec((RE, D), lambda i: (i, 0)),
            _full((3, D)), _full((1, D)), _full((1, D)), _full((1, D)),
            _full((D, D)), _full((1, D)),
        ],
        out_specs=(pl.BlockSpec((RE, H), lambda i: (i, 0)),
                   pl.BlockSpec((RE, H), lambda i: (i, 0))),
        out_shape=(jax.ShapeDtypeStruct((E, H), _F32),
                   jax.ShapeDtypeStruct((E, H), _F32)),
    )(ef, g, w1, b1, gl, bl, w2, b2)


def _edge_body(e0, e1, g, w1a, w1b, b1, gl, bl, w2, b2, o0, o1):
    x = (jnp.dot(e0[...], w1a[...], preferred_element_type=_F32)
         + jnp.dot(e1[...], w1b[...], preferred_element_type=_F32)
         + g[...] + b1[...])
    h = _ln(jnp.maximum(x, 0.0), gl[...], bl[...])
    y = jnp.dot(h, w2[...], preferred_element_type=_F32) + b2[...]
    o0[...] = y[:, :H]
    o1[...] = y[:, H:]


def _edge(e0, e1, g, w1a, w1b, b1, gl, bl, w2, b2):
    grid = (E // RE,)
    return pl.pallas_call(
        _edge_body,
        grid=grid,
        in_specs=[
            pl.BlockSpec((RE, H), lambda i: (i, 0)),
            pl.BlockSpec((RE, H), lambda i: (i, 0)),
            pl.BlockSpec((RE, D), lambda i: (i, 0)),
            _full((H, D)), _full((H, D)), _full((1, D)), _full((1, D)),
            _full((1, D)), _full((D, D)), _full((1, D)),
        ],
        out_specs=(pl.BlockSpec((RE, H), lambda i: (i, 0)),
                   pl.BlockSpec((RE, H), lambda i: (i, 0))),
        out_shape=(jax.ShapeDtypeStruct((E, H), _F32),
                   jax.ShapeDtypeStruct((E, H), _F32)),
    )(e0, e1, g, w1a, w1b, b1, gl, bl, w2, b2)


def _nsp0_body(m0, m1, w1a, w1b, b1, gl, bl, w2, b2, wr, s_out, a_out):
    x = (jnp.dot(m0[...], w1a[...], preferred_element_type=_F32)
         + jnp.dot(m1[...], w1b[...], preferred_element_type=_F32) + b1[...])
    h = _ln(jnp.maximum(x, 0.0), gl[...], bl[...])
    s = jnp.dot(h, w2[...], preferred_element_type=_F32) + b2[...]
    s_out[...] = s
    a_out[...] = jnp.dot(s, wr[...], preferred_element_type=_F32)


def _nsp0(m0, m1, w1a, w1b, b1, gl, bl, w2, b2, wr):
    grid = (N_SPATIAL // RN,)
    return pl.pallas_call(
        _nsp0_body,
        grid=grid,
        in_specs=[
            pl.BlockSpec((RN, H), lambda i: (i, 0)),
            pl.BlockSpec((RN, H), lambda i: (i, 0)),
            _full((H, D)), _full((H, D)), _full((1, D)), _full((1, D)),
            _full((1, D)), _full((D, D)), _full((1, D)), _full((D, D)),
        ],
        out_specs=(pl.BlockSpec((RN, D), lambda i: (i, 0)),
                   pl.BlockSpec((RN, D), lambda i: (i, 0))),
        out_shape=(jax.ShapeDtypeStruct((N_SPATIAL, D), _F32),
                   jax.ShapeDtypeStruct((N_SPATIAL, D), _F32)),
    )(m0, m1, w1a, w1b, b1, gl, bl, w2, b2, wr)


def _nsp_body(sp, m0, m1, w1t, w1a, w1b, b1, gl, bl, w2, b2, wr, s_out, a_out):
    x = (jnp.dot(sp[...], w1t[...], preferred_element_type=_F32)
         + jnp.dot(m0[...], w1a[...], preferred_element_type=_F32)
         + jnp.dot(m1[...], w1b[...], preferred_element_type=_F32) + b1[...])
    h = _ln(jnp.maximum(x, 0.0), gl[...], bl[...])
    s = jnp.dot(h, w2[...], preferred_element_type=_F32) + b2[...]
    s_out[...] = s
    a_out[...] = jnp.dot(s, wr[...], preferred_element_type=_F32)


def _nsp(sp, m0, m1, w1t, w1a, w1b, b1, gl, bl, w2, b2, wr):
    grid = (N_SPATIAL // RN,)
    return pl.pallas_call(
        _nsp_body,
        grid=grid,
        in_specs=[
            pl.BlockSpec((RN, D), lambda i: (i, 0)),
            pl.BlockSpec((RN, H), lambda i: (i, 0)),
            pl.BlockSpec((RN, H), lambda i: (i, 0)),
            _full((D, D)), _full((H, D)), _full((H, D)), _full((1, D)),
            _full((1, D)), _full((1, D)), _full((D, D)), _full((1, D)),
            _full((D, D)),
        ],
        out_specs=(pl.BlockSpec((RN, D), lambda i: (i, 0)),
                   pl.BlockSpec((RN, D), lambda i: (i, 0))),
        out_shape=(jax.ShapeDtypeStruct((N_SPATIAL, D), _F32),
                   jax.ShapeDtypeStruct((N_SPATIAL, D), _F32)),
    )(sp, m0, m1, w1t, w1a, w1b, b1, gl, bl, w2, b2, wr)


def _nsph_body(sn, w1t, b1, gl, bl, w2, b2, ws, s_out, a_out):
    x = jnp.dot(sn[...], w1t[...], preferred_element_type=_F32) + b1[...]
    h = _ln(jnp.maximum(x, 0.0), gl[...], bl[...])
    s = jnp.dot(h, w2[...], preferred_element_type=_F32) + b2[...]
    s_out[...] = s
    a_out[...] = jnp.dot(s, ws[...], preferred_element_type=_F32)


def _nsph(sn, w1t, b1, gl, bl, w2, b2, ws):
    return pl.pallas_call(
        _nsph_body,
        out_shape=(jax.ShapeDtypeStruct((N_SPHERE, D), _F32),
                   jax.ShapeDtypeStruct((N_SPHERE, D), _F32)),
    )(sn, w1t, b1, gl, bl, w2, b2, ws)


def _nfin_body(sp, m0, m1, w1t, w1a, w1b, b1, gl, bl, w2, b2,
               fw1, fb1, fgl, fbl, fw2, fb2, out):
    x = (jnp.dot(sp[...], w1t[...], preferred_element_type=_F32)
         + jnp.dot(m0[...], w1a[...], preferred_element_type=_F32)
         + jnp.dot(m1[...], w1b[...], preferred_element_type=_F32) + b1[...])
    h = _ln(jnp.maximum(x, 0.0), gl[...], bl[...])
    s = jnp.dot(h, w2[...], preferred_element_type=_F32) + b2[...]
    x2 = jnp.dot(s, fw1[...], preferred_element_type=_F32) + fb1[...]
    h2 = _ln(jnp.maximum(x2, 0.0), fgl[...], fbl[...])
    out[...] = jnp.dot(h2, fw2[...], preferred_element_type=_F32) + fb2[...]


def _nfin(sp, m0, m1, w1t, w1a, w1b, b1, gl, bl, w2, b2,
          fw1, fb1, fgl, fbl, fw2, fb2):
    grid = (N_SPATIAL // RN,)
    return pl.pallas_call(
        _nfin_body,
        grid=grid,
        in_specs=[
            pl.BlockSpec((RN, D), lambda i: (i, 0)),
            pl.BlockSpec((RN, H), lambda i: (i, 0)),
            pl.BlockSpec((RN, H), lambda i: (i, 0)),
            _full((D, D)), _full((H, D)), _full((H, D)), _full((1, D)),
            _full((1, D)), _full((1, D)), _full((D, D)), _full((1, D)),
            _full((D, D)), _full((1, D)), _full((1, D)), _full((1, D)),
            _full((D, NF)), _full((1, NF)),
        ],
        out_specs=pl.BlockSpec((RN, NF), lambda i: (i, 0)),
        out_shape=jax.ShapeDtypeStruct((N_SPATIAL, NF), _F32),
    )(sp, m0, m1, w1t, w1a, w1b, b1, gl, bl, w2, b2,
      fw1, fb1, fgl, fbl, fw2, fb2)


def _sc_mesh():
    return plsc.VectorSubcoreMesh(core_axis_name="c", subcore_axis_name="s",
                                  num_cores=NC, num_subcores=NS)


def _gather1(table, sidx3):
    @functools.partial(
        pl.kernel,
        out_type=jax.ShapeDtypeStruct((E, D), _F32),
        mesh=_sc_mesh(),
        scratch_types=[pltpu.VMEM((G_NCH, G_C), jnp.int32),
                       pltpu.VMEM((G_C, D), _F32)],
    )
    def k(tab_hbm, idx_hbm, out_hbm, idx_v, buf):
        wid = lax.axis_index("s") * NC + lax.axis_index("c")
        base = wid * G_EPW
        pltpu.sync_copy(idx_hbm.at[wid], idx_v)

        def chunk(kk, _):
            pltpu.sync_copy(tab_hbm.at[idx_v.at[kk]], buf)
            pltpu.sync_copy(buf, out_hbm.at[pl.ds(base + kk * G_C, G_C)])
            return 0

        lax.fori_loop(0, G_NCH, chunk, 0)

    return k(table, sidx3)


def _gather2(tab_s, tab_r, sidx3, ridx3):
    @functools.partial(
        pl.kernel,
        out_type=jax.ShapeDtypeStruct((E, D), _F32),
        mesh=_sc_mesh(),
        scratch_types=[pltpu.VMEM((G_NCH, G_C), jnp.int32),
                       pltpu.VMEM((G_NCH, G_C), jnp.int32),
                       pltpu.VMEM((G_C, D), _F32),
                       pltpu.VMEM((G_C, D), _F32)],
    )
    def k(ts_hbm, tr_hbm, sidx_hbm, ridx_hbm, out_hbm, sv, rv, ba, bb):
        wid = lax.axis_index("s") * NC + lax.axis_index("c")
        base = wid * G_EPW
        pltpu.sync_copy(sidx_hbm.at[wid], sv)
        pltpu.sync_copy(ridx_hbm.at[wid], rv)

        def chunk(kk, _):
            pltpu.sync_copy(ts_hbm.at[sv.at[kk]], ba)
            pltpu.sync_copy(tr_hbm.at[rv.at[kk]], bb)

            def addrow(r, _):
                for v in range(D // 16):
                    sl = pl.ds(v * 16, 16)
                    ba[r, sl] = ba[r, sl] + bb[r, sl]
                return 0

            lax.fori_loop(0, G_C, addrow, 0)
            pltpu.sync_copy(ba, out_hbm.at[pl.ds(base + kk * G_C, G_C)])
            return 0

        lax.fori_loop(0, G_NCH, chunk, 0)

    return k(tab_s, tab_r, sidx3, ridx3)


def _segsum(e0, e1, ridx3):
    @functools.partial(
        pl.kernel,
        out_type=(jax.ShapeDtypeStruct((N_SPATIAL, H), _F32),
                  jax.ShapeDtypeStruct((N_SPATIAL, H), _F32)),
        mesh=_sc_mesh(),
        scratch_types=[pltpu.VMEM((S_NCH, S_C), jnp.int32),
                       pltpu.VMEM((S_C, H), _F32),
                       pltpu.VMEM_SHARED((N_SPATIAL, H), _F32)],
    )
    def k(e0_hbm, e1_hbm, idx_hbm, o0_hbm, o1_hbm, idx_v, buf, acc):
        cid = lax.axis_index("c")
        sid = lax.axis_index("s")

        zv = jnp.zeros((16,), _F32)

        def zrow(r, _):
            for v in range(H // 16):
                buf[r, pl.ds(v * 16, 16)] = zv
            return 0

        lax.fori_loop(0, S_C, zrow, 0)
        zbase = sid * S_RPT
        for j in range(S_RPT // S_C):
            pltpu.sync_copy(buf, acc.at[pl.ds(zbase + j * S_C, S_C)])
        rem = S_RPT % S_C
        if rem:
            pltpu.sync_copy(buf.at[pl.ds(0, rem)],
                            acc.at[pl.ds(zbase + (S_RPT // S_C) * S_C, rem)])

        @pl.when(sid == NS - 1)
        def _():
            pltpu.sync_copy(buf.at[pl.ds(0, S_TAIL)],
                            acc.at[pl.ds(NS * S_RPT, S_TAIL)])

        plsc.subcore_barrier()

        pltpu.sync_copy(idx_hbm.at[sid], idx_v)
        base = sid * S_EPT

        def chunk(kk, _):
            off = base + kk * S_C

            @pl.when(cid == 0)
            def _():
                pltpu.sync_copy(e0_hbm.at[pl.ds(off, S_C)], buf)

            @pl.when(cid == 1)
            def _():
                pltpu.sync_copy(e1_hbm.at[pl.ds(off, S_C)], buf)

            pltpu.sync_copy(buf, acc.at[idx_v.at[kk]], add=True)
            return 0

        lax.fori_loop(0, S_NCH, chunk, 0)
        plsc.subcore_barrier()

        rows = pl.ds(sid * S_RPT, S_RPT)
        tail = pl.ds(NS * S_RPT, S_TAIL)
        is_last = sid == NS - 1

        @pl.when(cid == 0)
        def _():
            pltpu.sync_copy(acc.at[rows], o0_hbm.at[rows])

        @pl.when(jnp.logical_and(cid == 0, is_last))
        def _():
            pltpu.sync_copy(acc.at[tail], o0_hbm.at[tail])

        @pl.when(cid == 1)
        def _():
            pltpu.sync_copy(acc.at[rows], o1_hbm.at[rows])

        @pl.when(jnp.logical_and(cid == 1, is_last))
        def _():
            pltpu.sync_copy(acc.at[tail], o1_hbm.at[tail])

    return k(e0, e1, ridx3)


def kernel(sphere_nodes, edge_features, senders, receivers, params):
    p = params
    sidx3 = (senders - N_SPATIAL).reshape(NW, G_NCH, G_C)
    ridx3 = receivers.reshape(NW, G_NCH, G_C)
    ridx_t = receivers.reshape(NS, S_NCH, S_C)

    def row(v):
        return v.reshape(1, -1)

    sphere_proj, a_s0 = _prep(sphere_nodes, p['Wp'], p['eW1_0'][3:3 + D])
    g0 = _gather1(a_s0, sidx3)
    e0a, e0b = _edge0(edge_features, g0, p['eW1_0'][0:3], row(p['eb1_0']),
                      row(p['eg_0']), row(p['ebt_0']), p['eW2_0'], row(p['eb2_0']))
    m0a, m0b = _segsum(e0a, e0b, ridx_t)
    spatial, a_r = _nsp0(m0a, m0b, p['nW1_0'][D:D + H], p['nW1_0'][D + H:2 * D],
                         row(p['nb1_0']), row(p['ng_0']), row(p['nbt_0']),
                         p['nW2_0'], row(p['nb2_0']), p['eW1_1'][2 * D:3 * D])
    sphere, a_s = _nsph(sphere_proj, p['nW1_0'][0:D], row(p['nb1_0']),
                        row(p['ng_0']), row(p['nbt_0']), p['nW2_0'],
                        row(p['nb2_0']), p['eW1_1'][D:2 * D])

    g1 = _gather2(a_s, a_r, sidx3, ridx3)
    e1a, e1b = _edge(e0a, e0b, g1, p['eW1_1'][0:H], p['eW1_1'][H:D],
                     row(p['eb1_1']), row(p['eg_1']), row(p['ebt_1']),
                     p['eW2_1'], row(p['eb2_1']))
    m1a, m1b = _segsum(e1a, e1b, ridx_t)
    spatial, a_r = _nsp(spatial, m1a, m1b, p['nW1_1'][0:D],
                        p['nW1_1'][D:D + H], p['nW1_1'][D + H:2 * D],
                        row(p['nb1_1']), row(p['ng_1']), row(p['nbt_1']),
                        p['nW2_1'], row(p['nb2_1']), p['eW1_2'][2 * D:3 * D])
    _, a_s = _nsph(sphere, p['nW1_1'][0:D], row(p['nb1_1']),
                   row(p['ng_1']), row(p['nbt_1']), p['nW2_1'],
                   row(p['nb2_1']), p['eW1_2'][D:2 * D])

    g2 = _gather2(a_s, a_r, sidx3, ridx3)
    e2a, e2b = _edge(e1a, e1b, g2, p['eW1_2'][0:H], p['eW1_2'][H:D],
                     row(p['eb1_2']), row(p['eg_2']), row(p['ebt_2']),
                     p['eW2_2'], row(p['eb2_2']))
    m2a, m2b = _segsum(e2a, e2b, ridx_t)
    out = _nfin(spatial, m2a, m2b, p['nW1_2'][0:D],
                p['nW1_2'][D:D + H], p['nW1_2'][D + H:2 * D],
                row(p['nb1_2']), row(p['ng_2']), row(p['nbt_2']),
                p['nW2_2'], row(p['nb2_2']),
                p['fW1'], row(p['fb1']), row(p['fg']), row(p['fbt']),
                p['fW2'], row(p['fb2']))
    return out
